# baseline (speedup 1.0000x reference)
import functools

import jax
import jax.numpy as jnp
from jax import lax
from jax.experimental import pallas as pl
from jax.experimental.pallas import tpu as pltpu
from jax.experimental.pallas import tpu_sc as plsc

N = 10
L = 16

_mesh = plsc.VectorSubcoreMesh(
    core_axis_name="c", subcore_axis_name="s", num_cores=1, num_subcores=1
)


@functools.partial(
    pl.kernel,
    out_type=jax.ShapeDtypeStruct((1,), jnp.float32),
    mesh=_mesh,
    scratch_types=[
        pltpu.VMEM((L,), jnp.float32),
        pltpu.VMEM((L,), jnp.float32),
        pltpu.VMEM((L,), jnp.int32),
        pltpu.VMEM((L,), jnp.float32),
        pltpu.SemaphoreType.DMA,
        pltpu.SemaphoreType.DMA,
        pltpu.SemaphoreType.DMA,
    ],
    compiler_params=pltpu.CompilerParams(allow_input_fusion=[False, False, True]),
)
def _alpha_loss_sc(lb_hbm, ub_hbm, tgt_hbm, out_hbm, lb_v, ub_v, tgt_v, res_v,
                   sem0, sem1, sem2):
    c0 = pltpu.async_copy(lb_hbm.at[0], lb_v.at[pl.ds(0, N)], sem0)
    c1 = pltpu.async_copy(ub_hbm.at[0], ub_v.at[pl.ds(0, N)], sem1)
    c2 = pltpu.async_copy(tgt_hbm, tgt_v.at[pl.ds(0, 1)], sem2)
    c0.wait()
    c1.wait()
    c2.wait()

    lb = lb_v[...]
    ub = ub_v[...]
    tgt = tgt_v[...][0]
    lane = lax.iota(jnp.int32, L)

    h = jnp.where(lane < N, ub, 0.0)
    h = jnp.where(lane == tgt, jnp.float32(-(N - 1)) * lb, h)

    for k in (1, 2, 4, 8):
        h = h + h.at[lane ^ k].get(mode="promise_in_bounds")

    res_v[...] = h
    pltpu.sync_copy(res_v.at[pl.ds(0, 1)], out_hbm)


def kernel(lower_bounds, upper_bounds, target):
    tgt1 = jnp.reshape(jnp.asarray(target, dtype=jnp.int32), (1,))
    out = _alpha_loss_sc(lower_bounds, upper_bounds, tgt1)
    return jnp.reshape(out, ())


# speedup vs baseline: 1.0003x; 1.0003x over previous
import functools

import jax
import jax.numpy as jnp
from jax import lax
from jax.experimental import pallas as pl
from jax.experimental.pallas import tpu as pltpu
from jax.experimental.pallas import tpu_sc as plsc

N = 10
L = 16

_mesh = plsc.VectorSubcoreMesh(
    core_axis_name="c", subcore_axis_name="s", num_cores=1, num_subcores=1
)


@functools.partial(
    pl.kernel,
    out_type=jax.ShapeDtypeStruct((1,), jnp.float32),
    mesh=_mesh,
    scratch_types=[
        pltpu.VMEM((L,), jnp.float32),
        pltpu.VMEM((L,), jnp.float32),
        pltpu.VMEM((L,), jnp.int32),
        pltpu.VMEM((L,), jnp.float32),
        pltpu.SemaphoreType.DMA,
        pltpu.SemaphoreType.DMA,
        pltpu.SemaphoreType.DMA,
    ],
)
def _alpha_loss_sc(lb_hbm, ub_hbm, tgt_hbm, out_hbm, lb_v, ub_v, tgt_v, res_v,
                   sem0, sem1, sem2):
    c0 = pltpu.async_copy(lb_hbm.at[0], lb_v.at[pl.ds(0, N)], sem0)
    c1 = pltpu.async_copy(ub_hbm.at[0], ub_v.at[pl.ds(0, N)], sem1)
    c2 = pltpu.async_copy(tgt_hbm, tgt_v.at[pl.ds(0, 1)], sem2)
    c0.wait()
    c1.wait()
    c2.wait()

    lb = lb_v[...]
    ub = ub_v[...]
    tgt = tgt_v[...][0]
    lane = lax.iota(jnp.int32, L)

    h = jnp.where(lane < N, ub, 0.0)
    h = jnp.where(lane == tgt, jnp.float32(-(N - 1)) * lb, h)

    for k in (1, 2, 4, 8):
        h = h + h.at[lane ^ k].get(mode="promise_in_bounds")

    res_v[...] = h
    pltpu.sync_copy(res_v.at[pl.ds(0, 1)], out_hbm)


def kernel(lower_bounds, upper_bounds, target):
    tgt1 = jnp.reshape(jnp.asarray(target, dtype=jnp.int32), (1,))
    out = _alpha_loss_sc(lower_bounds, upper_bounds, tgt1)
    return jnp.reshape(out, ())


# speedup vs baseline: 1.0054x; 1.0051x over previous
import functools

import jax
import jax.numpy as jnp
from jax import lax
from jax.experimental import pallas as pl
from jax.experimental.pallas import tpu as pltpu
from jax.experimental.pallas import tpu_sc as plsc

N = 10
L = 16


@functools.cache
def _build_alpha_loss_sc():
    mesh = plsc.VectorSubcoreMesh(
        core_axis_name="c", subcore_axis_name="s", num_cores=1, num_subcores=1
    )

    @functools.partial(
        pl.kernel,
        out_type=jax.ShapeDtypeStruct((1,), jnp.float32),
        mesh=mesh,
        scratch_types=[
            pltpu.VMEM((L,), jnp.float32),
            pltpu.VMEM((L,), jnp.float32),
            pltpu.VMEM((L,), jnp.int32),
            pltpu.VMEM((L,), jnp.float32),
            pltpu.SemaphoreType.DMA,
            pltpu.SemaphoreType.DMA,
            pltpu.SemaphoreType.DMA,
        ],
    )
    def _alpha_loss_sc(lb_hbm, ub_hbm, tgt_hbm, out_hbm, lb_v, ub_v, tgt_v,
                       res_v, sem0, sem1, sem2):
        c0 = pltpu.async_copy(lb_hbm.at[0], lb_v.at[pl.ds(0, N)], sem0)
        c1 = pltpu.async_copy(ub_hbm.at[0], ub_v.at[pl.ds(0, N)], sem1)
        c2 = pltpu.async_copy(tgt_hbm, tgt_v.at[pl.ds(0, 1)], sem2)
        c0.wait()
        c1.wait()
        c2.wait()

        lb = lb_v[...]
        ub = ub_v[...]
        tgt = tgt_v[...][0]
        lane = lax.iota(jnp.int32, L)

        h = jnp.where(lane < N, ub, 0.0)
        h = jnp.where(lane == tgt, jnp.float32(-(N - 1)) * lb, h)

        for k in (1, 2, 4, 8):
            h = h + h.at[lane ^ k].get(mode="promise_in_bounds")

        res_v[...] = h
        pltpu.sync_copy(res_v.at[pl.ds(0, 1)], out_hbm)

    return _alpha_loss_sc


def kernel(lower_bounds, upper_bounds, target):
    tgt1 = jnp.reshape(jnp.asarray(target, dtype=jnp.int32), (1,))
    out = _build_alpha_loss_sc()(lower_bounds, upper_bounds, tgt1)
    return jnp.reshape(out, ())
